# Initial kernel scaffold; baseline (speedup 1.0000x reference)
import jax
import jax.numpy as jnp
from jax.experimental import pallas as pl


def kernel(x, e, edge_index, inverse_edge_index, U_w, U_b, Vf_w, Vf_b, Vt_w, Vt_b, iU_w, iU_b, W_placeholder):
    raise NotImplementedError("write your pallas kernel here")


# speedup vs baseline: 5.1042x; 5.1042x over previous
import functools

import jax
import jax.numpy as jnp
from jax import lax
from jax.experimental import pallas as pl
from jax.experimental.pallas import tpu as pltpu
from jax.experimental.pallas import tpu_sc as plsc

B, N, K, H = 2, 10000, 20, 128
E = N * K
BE = B * E
EBLK = 1600
NPB = EBLK // K
EB_PER_B = E // EBLK
BPB = EB_PER_B + 1
NBT = B * BPB
NB = B * EB_PER_B
EPAD = E + EBLK
TBLK = 2000

C = 128
NCH = BE // C


def _tables_body(x_ref, vfw_ref, vfb_ref, vtw_ref, vtb_ref, vf_ref, vt_ref):
    xb = x_ref[...]
    dn = (((1,), (1,)), ((), ()))
    vf_ref[...] = lax.dot_general(xb, vfw_ref[...], dn,
                                  preferred_element_type=jnp.float32) + vfb_ref[...]
    vt_ref[...] = lax.dot_general(xb, vtw_ref[...], dn,
                                  preferred_element_type=jnp.float32) + vtb_ref[...]


def _dense_body(e_ref, vf_ref, uw_ref, iuw_ref, consts_ref, base_ref, iue_ref):
    j = pl.program_id(0)
    p = j % BPB
    ph = p == EB_PER_B
    eb = e_ref[...]
    dn = (((1,), (1,)), ((), ()))
    ub = consts_ref[0:1, :]
    iub = consts_ref[1:2, :]
    plh = consts_ref[2:3, :]
    ue = lax.dot_general(eb, uw_ref[...], dn,
                         preferred_element_type=jnp.float32) + ub
    vf = vf_ref[...]
    vf_rep = jnp.broadcast_to(vf[:, None, :], (NPB, K, H)).reshape(EBLK, H)
    base_ref[...] = ue + vf_rep
    iue = lax.dot_general(eb, iuw_ref[...], dn,
                          preferred_element_type=jnp.float32) + iub
    iue_ref[...] = jnp.where(ph, jnp.broadcast_to(plh, (EBLK, H)), iue)


def _e_idx(j):
    b = j // BPB
    p = j % BPB
    return (b * EB_PER_B + jnp.minimum(p, EB_PER_B - 1), 0)


def _base_idx(j):
    b = j // BPB
    p = j % BPB
    real = b * EB_PER_B + jnp.minimum(p, EB_PER_B - 1)
    return (jnp.where(p == EB_PER_B, NB + b, real), 0)


_NW = 32


def _sc_gather_body(base_hbm, iue_hbm, vt_hbm, inv_hbm, edge_hbm, out_hbm,
                    inv_v, edge_v, r1_v, r2_v, base_v, out_v, sem1, sem2):
    wid = lax.axis_index("s") * 2 + lax.axis_index("c")
    nch_w = (NCH - wid + _NW - 1) // _NW

    def chunk_body(i, carry):
        ch = wid + i * _NW
        off = ch * C
        pltpu.sync_copy(inv_hbm.at[pl.ds(off, C)], inv_v)
        pltpu.sync_copy(edge_hbm.at[pl.ds(off, C)], edge_v)
        for t in range(C // 16):
            s = pl.ds(t * 16, 16)
            r = off + t * 16 + lax.iota(jnp.int32, 16)
            in_b1 = r >= E
            inv_v[s] = inv_v[s] + jnp.where(in_b1, EPAD, 0)
            edge_v[s] = edge_v[s] + jnp.where(in_b1, N, 0)
        cp1 = pltpu.async_copy(iue_hbm.at[inv_v], r1_v, sem1)
        cp2 = pltpu.async_copy(vt_hbm.at[edge_v], r2_v, sem2)
        pltpu.sync_copy(base_hbm.at[pl.ds(off, C)], base_v)
        cp1.wait()
        cp2.wait()

        def row_body(rr, c2):
            for t in range(H // 16):
                s = pl.ds(t * 16, 16)
                out_v[rr, s] = base_v[rr, s] + r1_v[rr, s] + r2_v[rr, s]
            return c2

        lax.fori_loop(0, C, row_body, 0)
        pltpu.sync_copy(out_v, out_hbm.at[pl.ds(off, C)])
        return carry

    lax.fori_loop(0, nch_w, chunk_body, 0)


def kernel(x, e, edge_index, inverse_edge_index, U_w, U_b, Vf_w, Vf_b,
           Vt_w, Vt_b, iU_w, iU_b, W_placeholder):
    x_flat = x.reshape(B * N, H)
    e_flat = e.reshape(BE, H)
    inv_flat = inverse_edge_index.reshape(BE)
    edge_flat = edge_index.reshape(BE)
    consts = jnp.concatenate(
        [U_b.reshape(1, H), iU_b.reshape(1, H), W_placeholder.reshape(1, H),
         jnp.zeros((5, H), jnp.float32)], axis=0)

    tbl_grid = (B * N) // TBLK
    vf_tab, vt_tab = pl.pallas_call(
        _tables_body,
        grid=(tbl_grid,),
        in_specs=[
            pl.BlockSpec((TBLK, H), lambda j: (j, 0)),
            pl.BlockSpec((H, H), lambda j: (0, 0)),
            pl.BlockSpec((1, H), lambda j: (0, 0)),
            pl.BlockSpec((H, H), lambda j: (0, 0)),
            pl.BlockSpec((1, H), lambda j: (0, 0)),
        ],
        out_specs=[
            pl.BlockSpec((TBLK, H), lambda j: (j, 0)),
            pl.BlockSpec((TBLK, H), lambda j: (j, 0)),
        ],
        out_shape=[
            jax.ShapeDtypeStruct((B * N, H), jnp.float32),
            jax.ShapeDtypeStruct((B * N, H), jnp.float32),
        ],
    )(x_flat, Vf_w, Vf_b.reshape(1, H), Vt_w, Vt_b.reshape(1, H))

    base_pad, iue_pad = pl.pallas_call(
        _dense_body,
        grid=(NBT,),
        in_specs=[
            pl.BlockSpec((EBLK, H), _e_idx),
            pl.BlockSpec((NPB, H), _e_idx),
            pl.BlockSpec((H, H), lambda j: (0, 0)),
            pl.BlockSpec((H, H), lambda j: (0, 0)),
            pl.BlockSpec((8, H), lambda j: (0, 0)),
        ],
        out_specs=[
            pl.BlockSpec((EBLK, H), _base_idx),
            pl.BlockSpec((EBLK, H), lambda j: (j, 0)),
        ],
        out_shape=[
            jax.ShapeDtypeStruct(((NB + B) * EBLK, H), jnp.float32),
            jax.ShapeDtypeStruct((NBT * EBLK, H), jnp.float32),
        ],
    )(e_flat, vf_tab, U_w, iU_w, consts)

    mesh = plsc.VectorSubcoreMesh(core_axis_name="c", subcore_axis_name="s",
                                  num_cores=2, num_subcores=16)
    out_flat = pl.kernel(
        _sc_gather_body,
        mesh=mesh,
        out_type=jax.ShapeDtypeStruct((BE, H), jnp.float32),
        scratch_types=[
            pltpu.VMEM((C,), jnp.int32),
            pltpu.VMEM((C,), jnp.int32),
            pltpu.VMEM((C, H), jnp.float32),
            pltpu.VMEM((C, H), jnp.float32),
            pltpu.VMEM((C, H), jnp.float32),
            pltpu.VMEM((C, H), jnp.float32),
            pltpu.SemaphoreType.DMA,
            pltpu.SemaphoreType.DMA,
        ],
    )(base_pad, iue_pad, vt_tab, inv_flat, edge_flat)

    return out_flat.reshape(B, E, H)


# speedup vs baseline: 6.6400x; 1.3009x over previous
import functools

import jax
import jax.numpy as jnp
from jax import lax
from jax.experimental import pallas as pl
from jax.experimental.pallas import tpu as pltpu
from jax.experimental.pallas import tpu_sc as plsc

B, N, K, H = 2, 10000, 20, 128
E = N * K
BE = B * E
EBLK = 1600
NPB = EBLK // K
EB_PER_B = E // EBLK
BPB = EB_PER_B + 1
NBT = B * BPB
NB = B * EB_PER_B
EPAD = E + EBLK
TBLK = 2000

C = 128
NCH = BE // C


def _tables_body(x_ref, vfw_ref, vfb_ref, vtw_ref, vtb_ref, vf_ref, vt_ref):
    xb = x_ref[...]
    dn = (((1,), (1,)), ((), ()))
    vf_ref[...] = lax.dot_general(xb, vfw_ref[...], dn,
                                  preferred_element_type=jnp.float32) + vfb_ref[...]
    vt_ref[...] = lax.dot_general(xb, vtw_ref[...], dn,
                                  preferred_element_type=jnp.float32) + vtb_ref[...]


def _dense_body(e_ref, vf_ref, uw_ref, iuw_ref, consts_ref, base_ref, iue_ref):
    j = pl.program_id(0)
    p = j % BPB
    ph = p == EB_PER_B
    eb = e_ref[...]
    dn = (((1,), (1,)), ((), ()))
    ub = consts_ref[0:1, :]
    iub = consts_ref[1:2, :]
    plh = consts_ref[2:3, :]
    ue = lax.dot_general(eb, uw_ref[...], dn,
                         preferred_element_type=jnp.float32) + ub
    vf = vf_ref[...]
    vf_rep = jnp.broadcast_to(vf[:, None, :], (NPB, K, H)).reshape(EBLK, H)
    base_ref[...] = ue + vf_rep
    iue = lax.dot_general(eb, iuw_ref[...], dn,
                          preferred_element_type=jnp.float32) + iub
    iue_ref[...] = jnp.where(ph, jnp.broadcast_to(plh, (EBLK, H)), iue)


def _e_idx(j):
    b = j // BPB
    p = j % BPB
    return (b * EB_PER_B + jnp.minimum(p, EB_PER_B - 1), 0)


def _base_idx(j):
    b = j // BPB
    p = j % BPB
    real = b * EB_PER_B + jnp.minimum(p, EB_PER_B - 1)
    return (jnp.where(p == EB_PER_B, NB + b, real), 0)


_NW = 32


NPT = 98
_NPAIR = NPT // 2


def _sc_gather_body(base_hbm, iue_hbm, vt_hbm, inv_hbm, edge_hbm, out_hbm,
                    inv0, inv1, edge0, edge1, r1a, r1b, r2a, r2b, acca, accb,
                    s_i0, s_i1, s_e0, s_e1, s_g10, s_g11, s_g20, s_g21,
                    s_b0, s_b1, s_o0, s_o1):
    wid = lax.axis_index("s") * 2 + lax.axis_index("c")

    def off_of(i):
        return jnp.minimum(wid + i * _NW, NCH - 1) * C

    def phase_a(i, inv_v, edge_v, s_i, s_e):
        off = off_of(i)
        pltpu.async_copy(inv_hbm.at[pl.ds(off, C)], inv_v, s_i)
        pltpu.async_copy(edge_hbm.at[pl.ds(off, C)], edge_v, s_e)

    def drain_out(acc_v, s_o):
        pltpu.make_async_copy(acc_v, out_hbm.at[pl.ds(0, C)], s_o).wait()

    def phase_b(i, inv_v, edge_v, s_i, s_e, r1_v, r2_v, acc_v, s_g1, s_g2,
                s_b):
        off = off_of(i)
        pltpu.make_async_copy(inv_hbm.at[pl.ds(off, C)], inv_v, s_i).wait()
        pltpu.make_async_copy(edge_hbm.at[pl.ds(off, C)], edge_v, s_e).wait()
        for t in range(C // 16):
            s = pl.ds(t * 16, 16)
            r = off + t * 16 + lax.iota(jnp.int32, 16)
            in_b1 = r >= E
            inv_v[s] = inv_v[s] + jnp.where(in_b1, EPAD, 0)
            edge_v[s] = edge_v[s] + jnp.where(in_b1, N, 0)
        pltpu.async_copy(iue_hbm.at[inv_v], r1_v, s_g1)
        pltpu.async_copy(vt_hbm.at[edge_v], r2_v, s_g2)
        pltpu.async_copy(base_hbm.at[pl.ds(off, C)], acc_v, s_b)

    def phase_c(i, inv_v, edge_v, r1_v, r2_v, acc_v, s_g1, s_g2, s_b, s_o):
        off = off_of(i)
        pltpu.make_async_copy(iue_hbm.at[inv_v], r1_v, s_g1).wait()
        pltpu.make_async_copy(vt_hbm.at[edge_v], r2_v, s_g2).wait()
        pltpu.make_async_copy(base_hbm.at[pl.ds(off, C)], acc_v, s_b).wait()

        def row_body(rr, c2):
            for t in range(H // 16):
                s = pl.ds(t * 16, 16)
                acc_v[rr, s] = acc_v[rr, s] + r1_v[rr, s] + r2_v[rr, s]
            return c2

        lax.fori_loop(0, C, row_body, 0)
        pltpu.async_copy(acc_v, out_hbm.at[pl.ds(off, C)], s_o)

    slot0 = (inv0, edge0, s_i0, s_e0, r1a, r2a, acca, s_g10, s_g20, s_b0)
    slot1 = (inv1, edge1, s_i1, s_e1, r1b, r2b, accb, s_g11, s_g21, s_b1)

    def b_(i, sl):
        phase_b(i, *sl)

    def c_(i, sl, s_o):
        inv_v, edge_v, _si, _se, r1_v, r2_v, acc_v, s_g1, s_g2, s_b = sl
        phase_c(i, inv_v, edge_v, r1_v, r2_v, acc_v, s_g1, s_g2, s_b, s_o)

    phase_a(0, inv0, edge0, s_i0, s_e0)
    phase_a(1, inv1, edge1, s_i1, s_e1)
    b_(0, slot0)

    def pair_body(p, carry):
        a = 2 * p

        @pl.when(p >= 1)
        def _():
            drain_out(accb, s_o1)

        b_(a + 1, slot1)
        c_(a, slot0, s_o0)
        phase_a(a + 2, inv0, edge0, s_i0, s_e0)
        phase_a(a + 3, inv1, edge1, s_i1, s_e1)
        c_(a + 1, slot1, s_o1)
        drain_out(acca, s_o0)
        b_(a + 2, slot0)
        return carry

    lax.fori_loop(0, _NPAIR - 1, pair_body, 0)

    last = NPT - 2
    drain_out(accb, s_o1)
    b_(last + 1, slot1)
    c_(last, slot0, s_o0)
    c_(last + 1, slot1, s_o1)
    drain_out(acca, s_o0)
    drain_out(accb, s_o1)


def kernel(x, e, edge_index, inverse_edge_index, U_w, U_b, Vf_w, Vf_b,
           Vt_w, Vt_b, iU_w, iU_b, W_placeholder):
    x_flat = x.reshape(B * N, H)
    e_flat = e.reshape(BE, H)
    inv_flat = inverse_edge_index.reshape(BE)
    edge_flat = edge_index.reshape(BE)
    consts = jnp.concatenate(
        [U_b.reshape(1, H), iU_b.reshape(1, H), W_placeholder.reshape(1, H),
         jnp.zeros((5, H), jnp.float32)], axis=0)

    tbl_grid = (B * N) // TBLK
    vf_tab, vt_tab = pl.pallas_call(
        _tables_body,
        grid=(tbl_grid,),
        in_specs=[
            pl.BlockSpec((TBLK, H), lambda j: (j, 0)),
            pl.BlockSpec((H, H), lambda j: (0, 0)),
            pl.BlockSpec((1, H), lambda j: (0, 0)),
            pl.BlockSpec((H, H), lambda j: (0, 0)),
            pl.BlockSpec((1, H), lambda j: (0, 0)),
        ],
        out_specs=[
            pl.BlockSpec((TBLK, H), lambda j: (j, 0)),
            pl.BlockSpec((TBLK, H), lambda j: (j, 0)),
        ],
        out_shape=[
            jax.ShapeDtypeStruct((B * N, H), jnp.float32),
            jax.ShapeDtypeStruct((B * N, H), jnp.float32),
        ],
    )(x_flat, Vf_w, Vf_b.reshape(1, H), Vt_w, Vt_b.reshape(1, H))

    base_pad, iue_pad = pl.pallas_call(
        _dense_body,
        grid=(NBT,),
        in_specs=[
            pl.BlockSpec((EBLK, H), _e_idx),
            pl.BlockSpec((NPB, H), _e_idx),
            pl.BlockSpec((H, H), lambda j: (0, 0)),
            pl.BlockSpec((H, H), lambda j: (0, 0)),
            pl.BlockSpec((8, H), lambda j: (0, 0)),
        ],
        out_specs=[
            pl.BlockSpec((EBLK, H), _base_idx),
            pl.BlockSpec((EBLK, H), lambda j: (j, 0)),
        ],
        out_shape=[
            jax.ShapeDtypeStruct(((NB + B) * EBLK, H), jnp.float32),
            jax.ShapeDtypeStruct((NBT * EBLK, H), jnp.float32),
        ],
    )(e_flat, vf_tab, U_w, iU_w, consts)

    mesh = plsc.VectorSubcoreMesh(core_axis_name="c", subcore_axis_name="s",
                                  num_cores=2, num_subcores=16)
    out_flat = pl.kernel(
        _sc_gather_body,
        mesh=mesh,
        out_type=jax.ShapeDtypeStruct((BE, H), jnp.float32),
        scratch_types=(
            [pltpu.VMEM((C,), jnp.int32)] * 4
            + [pltpu.VMEM((C, H), jnp.float32)] * 6
            + [pltpu.SemaphoreType.DMA] * 12
        ),
    )(base_pad, iue_pad, vt_tab, inv_flat, edge_flat)

    return out_flat.reshape(B, E, H)


# speedup vs baseline: 7.5308x; 1.1342x over previous
import functools

import jax
import jax.numpy as jnp
from jax import lax
from jax.experimental import pallas as pl
from jax.experimental.pallas import tpu as pltpu
from jax.experimental.pallas import tpu_sc as plsc

B, N, K, H = 2, 10000, 20, 128
E = N * K
BE = B * E
EBLK = 1600
NPB = EBLK // K
NB = BE // EBLK
TBLK = 2000

Q = 5
C = 128
NCH = BE // C
SCH = NCH // Q
SROWS = BE // Q
SBLK = NB // Q
_NW = 32
NPT = -(-SCH // _NW)
_NPAIR = NPT // 2


def _tables_body(x_ref, vfw_ref, vfb_ref, vtw_ref, vtb_ref, iuw_ref,
                 e01_ref, plc_ref, vf_ref, vt_ref, dummy_ref):
    j = pl.program_id(0)
    xb = x_ref[...]
    dn = (((1,), (1,)), ((), ()))
    vf_ref[...] = lax.dot_general(xb, vfw_ref[...], dn,
                                  preferred_element_type=jnp.float32) + vfb_ref[...]
    vt = lax.dot_general(xb, vtw_ref[...], dn,
                         preferred_element_type=jnp.float32) + vtb_ref[...]
    ip = lax.dot_general(e01_ref[...], iuw_ref[...], dn,
                         preferred_element_type=jnp.float32)
    shift = plc_ref[...] - ip
    row = jnp.where(j < 3 * (N // TBLK), shift[0:1, :], shift[1:2, :])
    vt_ref[...] = jnp.where(j < 2 * (N // TBLK), vt, vt + row)
    dummy_ref[...] = jnp.zeros((TBLK, H), jnp.float32)


def _sc_gather_body(stage, e_hbm, vt_hbm, inv_hbm, edge_hbm,
                    g1_hbm, g2_hbm,
                    inv0, inv1, edge0, edge1, r1a, r1b, r2a, r2b,
                    s_i0, s_i1, s_e0, s_e1, s_g10, s_g11, s_g20, s_g21,
                    s_o10, s_o11, s_o20, s_o21):
    wid = lax.axis_index("s") * 2 + lax.axis_index("c")

    def ch_of(i):
        return stage * SCH + jnp.minimum(wid + i * _NW, SCH - 1)

    def phase_a(i, inv_v, edge_v, s_i, s_e):
        off = ch_of(i) * C
        pltpu.async_copy(inv_hbm.at[pl.ds(off, C)], inv_v, s_i)
        pltpu.async_copy(edge_hbm.at[pl.ds(off, C)], edge_v, s_e)

    def phase_b(i, pred, inv_v, edge_v, s_i, s_e, r1_v, r2_v,
                s_g1, s_g2, s_o1, s_o2):
        off = ch_of(i) * C
        pltpu.make_async_copy(inv_hbm.at[pl.ds(off, C)], inv_v, s_i).wait()
        pltpu.make_async_copy(edge_hbm.at[pl.ds(off, C)], edge_v, s_e).wait()
        for t in range(C // 16):
            s = pl.ds(t * 16, 16)
            r = off + t * 16 + lax.iota(jnp.int32, 16)
            in_b1 = r >= E
            iv = inv_v[s]
            ph = iv == E
            inv_v[s] = jnp.where(ph, 0, iv) + jnp.where(in_b1, E, 0)
            edge_v[s] = (edge_v[s] + jnp.where(in_b1, N, 0)
                         + jnp.where(ph, 2 * N, 0))

        @pl.when(pred)
        def _():
            pltpu.make_async_copy(r1_v, g1_hbm.at[pl.ds(0, C)], s_o1).wait()
            pltpu.make_async_copy(r2_v, g2_hbm.at[pl.ds(0, C)], s_o2).wait()

        pltpu.async_copy(e_hbm.at[inv_v], r1_v, s_g1)
        pltpu.async_copy(vt_hbm.at[edge_v], r2_v, s_g2)

    def phase_c(i, inv_v, edge_v, r1_v, r2_v, s_g1, s_g2, s_o1, s_o2):
        off = ch_of(i) * C
        loc = off - stage * SROWS
        pltpu.make_async_copy(e_hbm.at[inv_v], r1_v, s_g1).wait()
        pltpu.make_async_copy(vt_hbm.at[edge_v], r2_v, s_g2).wait()
        pltpu.async_copy(r1_v, g1_hbm.at[pl.ds(loc, C)], s_o1)
        pltpu.async_copy(r2_v, g2_hbm.at[pl.ds(loc, C)], s_o2)

    slot0 = (inv0, edge0, s_i0, s_e0, r1a, r2a, s_g10, s_g20, s_o10, s_o20)
    slot1 = (inv1, edge1, s_i1, s_e1, r1b, r2b, s_g11, s_g21, s_o11, s_o21)

    def a_(i, sl):
        inv_v, edge_v, s_i, s_e = sl[0], sl[1], sl[2], sl[3]
        phase_a(i, inv_v, edge_v, s_i, s_e)

    def b_(i, pred, sl):
        inv_v, edge_v, s_i, s_e, r1_v, r2_v, s_g1, s_g2, s_o1, s_o2 = sl
        phase_b(i, pred, inv_v, edge_v, s_i, s_e, r1_v, r2_v,
                s_g1, s_g2, s_o1, s_o2)

    def c_(i, sl):
        inv_v, edge_v, _si, _se, r1_v, r2_v, s_g1, s_g2, s_o1, s_o2 = sl
        phase_c(i, inv_v, edge_v, r1_v, r2_v, s_g1, s_g2, s_o1, s_o2)

    a_(0, slot0)
    a_(1, slot1)
    b_(0, False, slot0)

    def pair_body(p, carry):
        a = 2 * p
        b_(a + 1, p >= 1, slot1)
        c_(a, slot0)
        a_(a + 2, slot0)
        a_(a + 3, slot1)
        c_(a + 1, slot1)
        b_(a + 2, True, slot0)
        return carry

    lax.fori_loop(0, _NPAIR - 1, pair_body, 0)

    last = NPT - 2
    b_(last + 1, True, slot1)
    c_(last, slot0)
    c_(last + 1, slot1)
    pltpu.make_async_copy(r1a, g1_hbm.at[pl.ds(0, C)], s_o10).wait()
    pltpu.make_async_copy(r2a, g2_hbm.at[pl.ds(0, C)], s_o20).wait()
    pltpu.make_async_copy(r1b, g1_hbm.at[pl.ds(0, C)], s_o11).wait()
    pltpu.make_async_copy(r2b, g2_hbm.at[pl.ds(0, C)], s_o21).wait()


def _apply_body(e_ref, g1_ref, g2_ref, vf_ref, uw_ref, iuw_ref,
                bias_ref, _alias_ref, out_ref):
    dn = (((1,), (1,)), ((), ()))
    ue = lax.dot_general(e_ref[...], uw_ref[...], dn,
                         preferred_element_type=jnp.float32)
    ig = lax.dot_general(g1_ref[...], iuw_ref[...], dn,
                         preferred_element_type=jnp.float32)
    vf = vf_ref[...]
    vf_rep = jnp.broadcast_to(vf[:, None, :], (NPB, K, H)).reshape(EBLK, H)
    out_ref[...] = ue + ig + g2_ref[...] + vf_rep + bias_ref[...]


def kernel(x, e, edge_index, inverse_edge_index, U_w, U_b, Vf_w, Vf_b,
           Vt_w, Vt_b, iU_w, iU_b, W_placeholder):
    x_flat = x.reshape(B * N, H)
    e_flat = e.reshape(BE, H)
    inv_flat = inverse_edge_index.reshape(BE)
    edge_flat = edge_index.reshape(BE)
    bias = (U_b + iU_b).reshape(1, H)
    plc = (W_placeholder - iU_b).reshape(1, H)
    plc2 = jnp.concatenate([plc, plc], axis=0)
    e01 = jnp.concatenate([e_flat[0:1], e_flat[E:E + 1]], axis=0)

    npb_blk = N // TBLK
    tbl_grid = 4 * npb_blk
    vf_tab, vt_ext, running = pl.pallas_call(
        _tables_body,
        grid=(tbl_grid,),
        in_specs=[
            pl.BlockSpec((TBLK, H), lambda j: (jnp.where(j < 10, j, j - 10), 0)),
            pl.BlockSpec((H, H), lambda j: (0, 0)),
            pl.BlockSpec((1, H), lambda j: (0, 0)),
            pl.BlockSpec((H, H), lambda j: (0, 0)),
            pl.BlockSpec((1, H), lambda j: (0, 0)),
            pl.BlockSpec((H, H), lambda j: (0, 0)),
            pl.BlockSpec((2, H), lambda j: (0, 0)),
            pl.BlockSpec((2, H), lambda j: (0, 0)),
        ],
        out_specs=[
            pl.BlockSpec((TBLK, H), lambda j: (jnp.where(j < 10, j, j - 10), 0)),
            pl.BlockSpec((TBLK, H), lambda j: (j, 0)),
            pl.BlockSpec((TBLK, H), lambda j: (0, 0)),
        ],
        out_shape=[
            jax.ShapeDtypeStruct((B * N, H), jnp.float32),
            jax.ShapeDtypeStruct((2 * B * N, H), jnp.float32),
            jax.ShapeDtypeStruct((BE, H), jnp.float32),
        ],
    )(x_flat, Vf_w, Vf_b.reshape(1, H), Vt_w, Vt_b.reshape(1, H),
      iU_w, e01, plc2)

    mesh = plsc.VectorSubcoreMesh(core_axis_name="c", subcore_axis_name="s",
                                  num_cores=2, num_subcores=16)
    gathered = []
    for s in range(Q):
        g1_s, g2_s = pl.kernel(
            functools.partial(_sc_gather_body, s),
            mesh=mesh,
            out_type=[
                jax.ShapeDtypeStruct((SROWS, H), jnp.float32),
                jax.ShapeDtypeStruct((SROWS, H), jnp.float32),
            ],
            scratch_types=(
                [pltpu.VMEM((C,), jnp.int32)] * 4
                + [pltpu.VMEM((C, H), jnp.float32)] * 4
                + [pltpu.SemaphoreType.DMA] * 12
            ),
        )(e_flat, vt_ext, inv_flat, edge_flat)
        gathered.append((g1_s, g2_s))

    for s in range(Q):
        g1_s, g2_s = gathered[s]
        running = pl.pallas_call(
            _apply_body,
            grid=(SBLK,),
            in_specs=[
                pl.BlockSpec((EBLK, H), functools.partial(_gidx, s)),
                pl.BlockSpec((EBLK, H), lambda j: (j, 0)),
                pl.BlockSpec((EBLK, H), lambda j: (j, 0)),
                pl.BlockSpec((NPB, H), functools.partial(_gidx, s)),
                pl.BlockSpec((H, H), lambda j: (0, 0)),
                pl.BlockSpec((H, H), lambda j: (0, 0)),
                pl.BlockSpec((1, H), lambda j: (0, 0)),
                pl.BlockSpec(memory_space=pl.ANY),
            ],
            out_specs=pl.BlockSpec((EBLK, H), functools.partial(_gidx, s)),
            out_shape=jax.ShapeDtypeStruct((BE, H), jnp.float32),
            input_output_aliases={7: 0},
        )(e_flat, g1_s, g2_s, vf_tab, U_w, iU_w, bias, running)

    return running.reshape(B, E, H)


def _gidx(s, j):
    return (s * SBLK + j, 0)


def _gidx3(s, j):
    return (s * SBLK + j, 0, 0)
